# baseline (speedup 1.0000x reference)
import functools

import jax
import jax.numpy as jnp
from jax import lax
from jax.experimental import pallas as pl
from jax.experimental.pallas import tpu as pltpu
from jax.experimental.pallas import tpu_sc as plsc

B = 16384
NUM_T = 1000
NUM_L = 16

NC = 2
NS = 16
L = 16
NW = NC * NS
BPW = B // NW
STEPS = BPW // L

_mesh = plsc.VectorSubcoreMesh(core_axis_name="c", subcore_axis_name="s")


@functools.partial(
    pl.kernel,
    out_type=jax.ShapeDtypeStruct((2 * B,), jnp.float32),
    mesh=_mesh,
    compiler_params=pltpu.CompilerParams(
        needs_layout_passes=False,
        disable_bounds_checks=True,
        disable_semaphore_checks=True,
    ),
    scratch_types=[
        pltpu.VMEM((BPW,), jnp.int32),
        pltpu.VMEM((BPW,), jnp.int32),
        pltpu.VMEM((NUM_T,), jnp.float32),
        pltpu.VMEM((NUM_L,), jnp.float32),
        pltpu.VMEM((2 * BPW,), jnp.float32),
        pltpu.SemaphoreType.DMA,
        pltpu.SemaphoreType.DMA,
        pltpu.SemaphoreType.DMA,
        pltpu.SemaphoreType.DMA,
    ],
)
def _encode(ts_hbm, ul_hbm, ttab_hbm, ltab_hbm, out_hbm,
            tidx_v, lidx_v, ttab_v, ltab_v, out_v,
            sem0, sem1, sem2, sem3):
    wid = lax.axis_index("s") * NC + lax.axis_index("c")
    base = wid * BPW

    cp0 = pltpu.async_copy(ttab_hbm, ttab_v, sem0)
    cp1 = pltpu.async_copy(ltab_hbm, ltab_v, sem1)
    cp2 = pltpu.async_copy(ts_hbm.at[pl.ds(base, BPW)], tidx_v, sem2)
    cp3 = pltpu.async_copy(ul_hbm.at[pl.ds(base, BPW)], lidx_v, sem3)
    cp0.wait()
    cp1.wait()
    cp2.wait()
    cp3.wait()

    lane2 = lax.iota(jnp.int32, L) * 2
    for i in range(STEPS):
        ti = tidx_v[pl.ds(i * L, L)]
        li = lidx_v[pl.ds(i * L, L)]
        tv = plsc.load_gather(ttab_v, [ti])
        lv = plsc.load_gather(ltab_v, [li])
        pos = lane2 + (2 * i * L)
        plsc.store_scatter(out_v, [pos], tv)
        plsc.store_scatter(out_v, [pos + 1], lv)

    pltpu.sync_copy(out_v, out_hbm.at[pl.ds(2 * base, 2 * BPW)])


def kernel(timestep, unet_layer, normalized_timesteps, normalized_unet_layers):
    flat = _encode(timestep, unet_layer, normalized_timesteps,
                   normalized_unet_layers)
    return flat.reshape(B, 2)


# speedup vs baseline: 1.2898x; 1.2898x over previous
import functools

import jax
import jax.numpy as jnp
from jax import lax
from jax.experimental import pallas as pl
from jax.experimental.pallas import tpu as pltpu
from jax.experimental.pallas import tpu_sc as plsc

B = 16384
NUM_T = 1000
NUM_L = 16

NC = 2
NS = 16
L = 16
NW = NC * NS
BPW = B // NW
STEPS = BPW // L

_mesh = plsc.VectorSubcoreMesh(core_axis_name="c", subcore_axis_name="s")


@functools.partial(
    pl.kernel,
    out_type=jax.ShapeDtypeStruct((B, 2), jnp.float32),
    mesh=_mesh,
    compiler_params=pltpu.CompilerParams(
        needs_layout_passes=False,
        disable_bounds_checks=True,
        disable_semaphore_checks=True,
    ),
    scratch_types=[
        pltpu.VMEM((BPW,), jnp.int32),
        pltpu.VMEM((BPW,), jnp.int32),
        pltpu.VMEM((NUM_T,), jnp.float32),
        pltpu.VMEM((NUM_L,), jnp.float32),
        pltpu.VMEM((BPW, 2), jnp.float32),
        pltpu.SemaphoreType.DMA,
        pltpu.SemaphoreType.DMA,
        pltpu.SemaphoreType.DMA,
        pltpu.SemaphoreType.DMA,
    ],
)
def _encode(ts_hbm, ul_hbm, ttab_hbm, ltab_hbm, out_hbm,
            tidx_v, lidx_v, ttab_v, ltab_v, out_v,
            sem0, sem1, sem2, sem3):
    wid = lax.axis_index("s") * NC + lax.axis_index("c")
    base = wid * BPW

    cp0 = pltpu.async_copy(ttab_hbm, ttab_v, sem0)
    cp1 = pltpu.async_copy(ltab_hbm, ltab_v, sem1)
    cp2 = pltpu.async_copy(ts_hbm.at[pl.ds(base, BPW)], tidx_v, sem2)
    cp3 = pltpu.async_copy(ul_hbm.at[pl.ds(base, BPW)], lidx_v, sem3)
    cp0.wait()
    cp1.wait()
    cp2.wait()
    cp3.wait()

    lane = lax.iota(jnp.int32, L)
    col0 = jnp.zeros((L,), jnp.int32)
    col1 = jnp.ones((L,), jnp.int32)

    @plsc.parallel_loop(0, BPW, step=L, unroll=4)
    def _(i):
        ti = tidx_v[pl.ds(i, L)]
        li = lidx_v[pl.ds(i, L)]
        tv = plsc.load_gather(ttab_v, [ti])
        lv = plsc.load_gather(ltab_v, [li])
        row = lane + i
        plsc.store_scatter(out_v, [row, col0], tv)
        plsc.store_scatter(out_v, [row, col1], lv)

    pltpu.sync_copy(out_v, out_hbm.at[pl.ds(base, BPW)])


def kernel(timestep, unet_layer, normalized_timesteps, normalized_unet_layers):
    return _encode(timestep, unet_layer, normalized_timesteps,
                   normalized_unet_layers)
